# baseline (speedup 1.0000x reference)
import jax
import jax.numpy as jnp
from jax import lax
from jax.experimental import pallas as pl
from jax.experimental.pallas import tpu as pltpu
from jax.experimental.pallas import tpu_sc as plsc

_H = 32
_NH = 12
_NB = 4
_S = _H * _H
_PW = 32
_NT = _NH * _NB


def _sc_body(tf_hbm, x_hbm, out_hbm, table_v, buf0, buf1, buf2,
             si0, si1, si2, so0, so1, so2):
    w = lax.axis_index("s") * 2 + lax.axis_index("c")
    bufs = (buf0, buf1, buf2)
    sin = (si0, si1, si2)
    sout = (so0, so1, so2)

    def x_slab(t):
        return x_hbm.at[t % _NB, t // _NB, pl.ds(w * _PW, _PW)]

    def out_slab(t):
        return out_hbm.at[t % _NB, t // _NB, pl.ds(w * _PW, _PW)]

    pltpu.async_copy(x_slab(0), bufs[0], sin[0])
    pltpu.async_copy(x_slab(1), bufs[1], sin[1])
    pltpu.sync_copy(tf_hbm.at[0], table_v)

    def super_body(s, carry):
        for k in range(3):
            t = 3 * s + k
            h = t // _NB

            @pl.when(t % _NB == 0)
            def _load_table():
                pltpu.sync_copy(tf_hbm.at[h], table_v)

            pltpu.make_async_copy(x_slab(0), bufs[k], sin[k]).wait()

            def add_body(pj, c):
                buf = bufs[k]
                for qi in range(_H):
                    a = w + (_H - 1) - qi
                    for half in range(2):
                        slq = pl.ds(qi * _H + 16 * half, 16)
                        ts = pl.ds(_H - 1 - pj + 16 * half, 16)
                        buf[pj, slq] = buf[pj, slq] + table_v[a, ts]
                return c

            lax.fori_loop(0, _PW, add_body, 0)
            pltpu.async_copy(bufs[k], out_slab(t), sout[k])

            kn = (k + 2) % 3

            @pl.when(t + 2 < _NT)
            def _prefetch():
                @pl.when(t >= 1)
                def _drain_prev_out():
                    pltpu.make_async_copy(bufs[kn], out_slab(0), sout[kn]).wait()
                pltpu.async_copy(x_slab(t + 2), bufs[kn], sin[kn])
        return carry

    lax.fori_loop(0, _NT // 3, super_body, 0)
    for k in range(3):
        pltpu.make_async_copy(bufs[k], out_slab(0), sout[k]).wait()


def kernel(x, relative_pos):
    tf = jnp.pad(relative_pos[:, :, ::-1], ((0, 0), (0, 1), (0, 1)))
    mesh = plsc.VectorSubcoreMesh(core_axis_name="c", subcore_axis_name="s",
                                  num_cores=2)
    run = pl.kernel(
        _sc_body,
        mesh=mesh,
        out_type=jax.ShapeDtypeStruct(x.shape, x.dtype),
        scratch_types=[
            pltpu.VMEM((64, 64), jnp.float32),
            pltpu.VMEM((_PW, _S), jnp.float32),
            pltpu.VMEM((_PW, _S), jnp.float32),
            pltpu.VMEM((_PW, _S), jnp.float32),
            pltpu.SemaphoreType.DMA,
            pltpu.SemaphoreType.DMA,
            pltpu.SemaphoreType.DMA,
            pltpu.SemaphoreType.DMA,
            pltpu.SemaphoreType.DMA,
            pltpu.SemaphoreType.DMA,
        ],
    )
    return run(tf, x)


# speedup vs baseline: 2.0257x; 2.0257x over previous
import jax
import jax.numpy as jnp
from jax.experimental import pallas as pl
from jax.experimental.pallas import tpu as pltpu

_H = 32
_NH = 12
_S = _H * _H
_M = 2 * _H - 1


def _fused_body(rp_ref, x_ref, o_ref, bias_ref):
    @pl.when(pl.program_id(1) == 0)
    def _build_bias():
        rp64 = jnp.pad(rp_ref[0], ((0, 1), (0, 1)))
        r = jax.lax.broadcasted_iota(jnp.int32, (_S, 64), 0)
        a = jax.lax.broadcasted_iota(jnp.int32, (_S, 64), 1)
        oi = (a == (r // _H - r % _H + (_H - 1))).astype(jnp.float32)
        c = jax.lax.broadcasted_iota(jnp.int32, (64, _S), 1)
        b = jax.lax.broadcasted_iota(jnp.int32, (64, _S), 0)
        ojt = (b == (c // _H - c % _H + (_H - 1))).astype(jnp.float32)
        t1 = jnp.dot(oi, rp64, preferred_element_type=jnp.float32)
        t2 = jnp.dot(t1, ojt, preferred_element_type=jnp.float32)
        t4 = t2.reshape(_H, _H, _H, _H).transpose(0, 2, 1, 3)
        bias_ref[...] = t4.reshape(_S, _S)

    o_ref[0, 0] = x_ref[0, 0] + bias_ref[...]


def kernel(x, relative_pos):
    return pl.pallas_call(
        _fused_body,
        grid=(_NH, x.shape[0]),
        in_specs=[
            pl.BlockSpec((1, _M, _M), lambda h, b: (h, 0, 0)),
            pl.BlockSpec((1, 1, _S, _S), lambda h, b: (b, h, 0, 0)),
        ],
        out_specs=pl.BlockSpec((1, 1, _S, _S), lambda h, b: (b, h, 0, 0)),
        out_shape=jax.ShapeDtypeStruct(x.shape, x.dtype),
        scratch_shapes=[pltpu.VMEM((_S, _S), jnp.float32)],
    )(relative_pos, x)
